# Initial kernel scaffold; baseline (speedup 1.0000x reference)
import jax
import jax.numpy as jnp
from jax.experimental import pallas as pl


def kernel(query, memory, adj_indices, W1, b1, W2, b2, W3, b3):
    raise NotImplementedError("write your pallas kernel here")


# speedup vs baseline: 2.5990x; 2.5990x over previous
import functools

import jax
import jax.numpy as jnp
from jax import lax
from jax.experimental import pallas as pl
from jax.experimental.pallas import tpu as pltpu
from jax.experimental.pallas import tpu_sc as plsc

NC = 2
NS = 16
NW = NC * NS
CH = 128


def _make_gather(E, N, D):
  n_chunks = E // CH
  iters = (n_chunks + NW - 1) // NW
  mesh = plsc.VectorSubcoreMesh(core_axis_name="c", subcore_axis_name="s")

  @functools.partial(
      pl.kernel, mesh=mesh,
      out_type=(jax.ShapeDtypeStruct((E, D), jnp.float32),
                jax.ShapeDtypeStruct((E, D), jnp.float32)),
      scratch_types=[
          pltpu.VMEM((CH,), jnp.int32),
          pltpu.VMEM((CH,), jnp.int32),
          pltpu.VMEM((CH, D), jnp.float32),
          pltpu.VMEM((CH, D), jnp.float32),
          pltpu.SemaphoreType.DMA,
          pltpu.SemaphoreType.DMA,
      ])
  def gather_k(query_hbm, memory_hbm, row_hbm, col_hbm, qg_hbm, vg_hbm,
               ridx, cidx, qbuf, vbuf, sem_q, sem_v):
    wid = lax.axis_index("s") * NC + lax.axis_index("c")

    def body(i, carry):
      c = wid + NW * i

      @pl.when(c < n_chunks)
      def _():
        base = c * CH
        pltpu.sync_copy(row_hbm.at[pl.ds(base, CH)], ridx)
        pltpu.sync_copy(col_hbm.at[pl.ds(base, CH)], cidx)
        cp_q = pltpu.async_copy(query_hbm.at[ridx], qbuf, sem_q)
        cp_v = pltpu.async_copy(memory_hbm.at[cidx], vbuf, sem_v)
        cp_q.wait()
        cp_v.wait()
        pltpu.sync_copy(qbuf, qg_hbm.at[pl.ds(base, CH)])
        pltpu.sync_copy(vbuf, vg_hbm.at[pl.ds(base, CH)])

      return carry

    lax.fori_loop(0, iters, body, 0)

  return gather_k


def _make_scatter(E, N, D):
  n_chunks = E // CH
  iters = (n_chunks + NW - 1) // NW
  mesh = plsc.VectorSubcoreMesh(core_axis_name="c", subcore_axis_name="s")

  @functools.partial(
      pl.kernel, mesh=mesh,
      out_type=jax.ShapeDtypeStruct((NC, N, D), jnp.float32),
      scratch_types=[
          pltpu.VMEM((CH,), jnp.int32),
          pltpu.VMEM((CH, D), jnp.float32),
          pltpu.VMEM_SHARED((N, D), jnp.float32),
      ])
  def scatter_k(ct_hbm, row_hbm, zero_hbm, out_hbm, idx, buf, acc):
    cid = lax.axis_index("c")
    sid = lax.axis_index("s")
    wid = sid * NC + cid

    @pl.when(sid == 0)
    def _():
      pltpu.sync_copy(zero_hbm, acc)

    plsc.subcore_barrier()

    def body(i, carry):
      c = wid + NW * i

      @pl.when(c < n_chunks)
      def _():
        base = c * CH
        pltpu.sync_copy(row_hbm.at[pl.ds(base, CH)], idx)
        pltpu.sync_copy(ct_hbm.at[pl.ds(base, CH)], buf)
        pltpu.sync_copy(buf, acc.at[idx], add=True)

      return carry

    lax.fori_loop(0, iters, body, 0)
    plsc.subcore_barrier()

    @pl.when(sid == 0)
    def _():
      pltpu.sync_copy(acc, out_hbm.at[cid])

  return scatter_k


def _mlp_body(qg, vg, w1a, w1b, w2, b1, b2, w3r, b3, ct):
  q = qg[...]
  v = vg[...]
  h = jnp.dot(q, w1a[...], preferred_element_type=jnp.float32)
  h += jnp.dot(v, w1b[...], preferred_element_type=jnp.float32)
  h = jnp.maximum(h + b1[...], 0.0)
  h = jnp.dot(h, w2[...], preferred_element_type=jnp.float32)
  h = jnp.maximum(h + b2[...], 0.0)
  logit = jnp.sum(h * w3r[...], axis=1, keepdims=True) + b3[...]
  w = jax.nn.sigmoid(logit)
  ct[...] = w * v


def _run_mlp(qg, vg, W1a, W1b, W2, b1, b2, w3r, b3, E, D, H, TE):
  grid = (E // TE,)
  const = lambda i: (0, 0)
  return pl.pallas_call(
      _mlp_body,
      grid=grid,
      in_specs=[
          pl.BlockSpec((TE, D), lambda i: (i, 0)),
          pl.BlockSpec((TE, D), lambda i: (i, 0)),
          pl.BlockSpec((D, H), const),
          pl.BlockSpec((D, H), const),
          pl.BlockSpec((H, H), const),
          pl.BlockSpec((1, H), const),
          pl.BlockSpec((1, H), const),
          pl.BlockSpec((1, H), const),
          pl.BlockSpec((1, 1), const),
      ],
      out_specs=pl.BlockSpec((TE, D), lambda i: (i, 0)),
      out_shape=jax.ShapeDtypeStruct((E, D), jnp.float32),
  )(qg, vg, W1a, W1b, W2, b1, b2, w3r, b3)


def _add_body(a, b, o):
  o[...] = a[...] + b[...]


def _run_add(p0, p1, N, D, TN):
  return pl.pallas_call(
      _add_body,
      grid=(N // TN,),
      in_specs=[pl.BlockSpec((TN, D), lambda i: (i, 0)),
                pl.BlockSpec((TN, D), lambda i: (i, 0))],
      out_specs=pl.BlockSpec((TN, D), lambda i: (i, 0)),
      out_shape=jax.ShapeDtypeStruct((N, D), jnp.float32),
  )(p0, p1)


def kernel(query, memory, adj_indices, W1, b1, W2, b2, W3, b3):
  N, D = query.shape
  E = adj_indices.shape[0]
  H = W2.shape[0]

  row = adj_indices[:, 0]
  col = adj_indices[:, 1]

  qg, vg = _make_gather(E, N, D)(query, memory, row, col)

  W1a = W1[:D]
  W1b = W1[D:]
  b1r = b1.reshape(1, H)
  b2r = b2.reshape(1, H)
  w3r = W3.reshape(1, H)
  b3r = b3.reshape(1, 1)
  ct = _run_mlp(qg, vg, W1a, W1b, W2, b1r, b2r, w3r, b3r, E, D, H, TE=1280)

  zeros = jnp.zeros((N, D), jnp.float32)
  parts = _make_scatter(E, N, D)(ct, row, zeros)

  return _run_add(parts[0], parts[1], N, D, TN=2000)


# speedup vs baseline: 2.6453x; 1.0178x over previous
import functools

import jax
import jax.numpy as jnp
from jax import lax
from jax.experimental import pallas as pl
from jax.experimental.pallas import tpu as pltpu
from jax.experimental.pallas import tpu_sc as plsc

NC = 2
NS = 16
NW = NC * NS
CH = 128


def _make_gather(E, N, D):
  n_chunks = E // CH
  iters = (n_chunks + NW - 1) // NW
  mesh = plsc.VectorSubcoreMesh(core_axis_name="c", subcore_axis_name="s")

  @functools.partial(
      pl.kernel, mesh=mesh,
      out_type=(jax.ShapeDtypeStruct((E, D), jnp.float32),
                jax.ShapeDtypeStruct((E, D), jnp.float32)),
      scratch_types=[
          pltpu.VMEM((CH,), jnp.int32),
          pltpu.VMEM((CH,), jnp.int32),
          pltpu.VMEM((CH, D), jnp.float32),
          pltpu.VMEM((CH, D), jnp.float32),
          pltpu.SemaphoreType.DMA,
          pltpu.SemaphoreType.DMA,
      ])
  def gather_k(query_hbm, memory_hbm, row_hbm, col_hbm, qg_hbm, vg_hbm,
               ridx, cidx, qbuf, vbuf, sem_q, sem_v):
    wid = lax.axis_index("s") * NC + lax.axis_index("c")

    def body(i, carry):
      c = wid + NW * i

      @pl.when(c < n_chunks)
      def _():
        base = c * CH
        pltpu.sync_copy(row_hbm.at[pl.ds(base, CH)], ridx)
        pltpu.sync_copy(col_hbm.at[pl.ds(base, CH)], cidx)
        cp_q = pltpu.async_copy(query_hbm.at[ridx], qbuf, sem_q)
        cp_v = pltpu.async_copy(memory_hbm.at[cidx], vbuf, sem_v)
        cp_q.wait()
        cp_v.wait()
        pltpu.sync_copy(qbuf, qg_hbm.at[pl.ds(base, CH)])
        pltpu.sync_copy(vbuf, vg_hbm.at[pl.ds(base, CH)])

      return carry

    lax.fori_loop(0, iters, body, 0)

  return gather_k


def _make_scatter(E, N, D):
  n_chunks = E // CH
  iters = (n_chunks + NW - 1) // NW
  mesh = plsc.VectorSubcoreMesh(core_axis_name="c", subcore_axis_name="s")

  @functools.partial(
      pl.kernel, mesh=mesh,
      out_type=jax.ShapeDtypeStruct((NC, N, D), jnp.float32),
      scratch_types=[
          pltpu.VMEM((CH,), jnp.int32),
          pltpu.VMEM((CH, D), jnp.float32),
          pltpu.VMEM_SHARED((N, D), jnp.float32),
      ])
  def scatter_k(ct_hbm, row_hbm, zero_hbm, out_hbm, idx, buf, acc):
    cid = lax.axis_index("c")
    sid = lax.axis_index("s")
    wid = sid * NC + cid

    @pl.when(sid == 0)
    def _():
      pltpu.sync_copy(zero_hbm, acc)

    plsc.subcore_barrier()

    def body(i, carry):
      c = wid + NW * i

      @pl.when(c < n_chunks)
      def _():
        base = c * CH
        pltpu.sync_copy(row_hbm.at[pl.ds(base, CH)], idx)
        pltpu.sync_copy(ct_hbm.at[pl.ds(base, CH)], buf)
        pltpu.sync_copy(buf, acc.at[idx], add=True)

      return carry

    lax.fori_loop(0, iters, body, 0)
    plsc.subcore_barrier()

    @pl.when(sid == 0)
    def _():
      pltpu.sync_copy(acc, out_hbm.at[cid])

  return scatter_k


def _mlp_body(qg, vg, w1a, w1b, w2, b1, b2, w3r, b3, ct):
  q = qg[...].astype(jnp.bfloat16)
  v = vg[...]
  h = jnp.dot(q, w1a[...], preferred_element_type=jnp.float32)
  h += jnp.dot(v.astype(jnp.bfloat16), w1b[...],
               preferred_element_type=jnp.float32)
  h = jnp.maximum(h + b1[...], 0.0)
  h = jnp.dot(h.astype(jnp.bfloat16), w2[...],
              preferred_element_type=jnp.float32)
  h = jnp.maximum(h + b2[...], 0.0)
  logit = jnp.sum(h * w3r[...], axis=1, keepdims=True) + b3[...]
  w = jax.nn.sigmoid(logit)
  ct[...] = w * v


def _run_mlp(qg, vg, W1a, W1b, W2, b1, b2, w3r, b3, E, D, H, TE):
  grid = (E // TE,)
  const = lambda i: (0, 0)
  return pl.pallas_call(
      _mlp_body,
      grid=grid,
      in_specs=[
          pl.BlockSpec((TE, D), lambda i: (i, 0)),
          pl.BlockSpec((TE, D), lambda i: (i, 0)),
          pl.BlockSpec((D, H), const),
          pl.BlockSpec((D, H), const),
          pl.BlockSpec((H, H), const),
          pl.BlockSpec((1, H), const),
          pl.BlockSpec((1, H), const),
          pl.BlockSpec((1, H), const),
          pl.BlockSpec((1, 1), const),
      ],
      out_specs=pl.BlockSpec((TE, D), lambda i: (i, 0)),
      out_shape=jax.ShapeDtypeStruct((E, D), jnp.float32),
  )(qg, vg, W1a, W1b, W2, b1, b2, w3r, b3)


def _add_body(a, b, o):
  o[...] = a[...] + b[...]


def _run_add(p0, p1, N, D, TN):
  return pl.pallas_call(
      _add_body,
      grid=(N // TN,),
      in_specs=[pl.BlockSpec((TN, D), lambda i: (i, 0)),
                pl.BlockSpec((TN, D), lambda i: (i, 0))],
      out_specs=pl.BlockSpec((TN, D), lambda i: (i, 0)),
      out_shape=jax.ShapeDtypeStruct((N, D), jnp.float32),
  )(p0, p1)


def kernel(query, memory, adj_indices, W1, b1, W2, b2, W3, b3):
  N, D = query.shape
  E = adj_indices.shape[0]
  H = W2.shape[0]

  row = adj_indices[:, 0]
  col = adj_indices[:, 1]

  qg, vg = _make_gather(E, N, D)(query, memory, row, col)

  W1a = W1[:D].astype(jnp.bfloat16)
  W1b = W1[D:].astype(jnp.bfloat16)
  b1r = b1.reshape(1, H)
  b2r = b2.reshape(1, H)
  w3r = W3.reshape(1, H)
  b3r = b3.reshape(1, 1)
  ct = _run_mlp(qg, vg, W1a, W1b, W2.astype(jnp.bfloat16), b1r, b2r, w3r,
                b3r, E, D, H, TE=1280)

  zeros = jnp.zeros((N, D), jnp.float32)
  parts = _make_scatter(E, N, D)(ct, row, zeros)

  return _run_add(parts[0], parts[1], N, D, TN=2000)


# speedup vs baseline: 3.2062x; 1.2121x over previous
import functools

import jax
import jax.numpy as jnp
from jax import lax
from jax.experimental import pallas as pl
from jax.experimental.pallas import tpu as pltpu
from jax.experimental.pallas import tpu_sc as plsc

NC = 2
NS = 16
NW = NC * NS
CH = 128


def _make_gather(E, N, D):
  n_chunks = E // CH
  iters = (n_chunks + NW - 1) // NW
  mesh = plsc.VectorSubcoreMesh(core_axis_name="c", subcore_axis_name="s")

  @functools.partial(
      pl.kernel, mesh=mesh,
      out_type=(jax.ShapeDtypeStruct((E, D), jnp.float32),
                jax.ShapeDtypeStruct((E, D), jnp.float32)),
      scratch_types=[
          pltpu.VMEM((CH,), jnp.int32),
          pltpu.VMEM((CH,), jnp.int32),
          pltpu.VMEM((CH, D), jnp.float32),
          pltpu.VMEM((CH, D), jnp.float32),
          pltpu.SemaphoreType.DMA,
          pltpu.SemaphoreType.DMA,
      ])
  def gather_k(query_hbm, memory_hbm, row_hbm, col_hbm, qg_hbm, vg_hbm,
               ridx, cidx, qbuf, vbuf, sem_q, sem_v):
    wid = lax.axis_index("s") * NC + lax.axis_index("c")

    def body(i, carry):
      c = wid + NW * i

      @pl.when(c < n_chunks)
      def _():
        base = c * CH
        pltpu.sync_copy(row_hbm.at[pl.ds(base, CH)], ridx)
        pltpu.sync_copy(col_hbm.at[pl.ds(base, CH)], cidx)
        cp_q = pltpu.async_copy(query_hbm.at[ridx], qbuf, sem_q)
        cp_v = pltpu.async_copy(memory_hbm.at[cidx], vbuf, sem_v)
        cp_q.wait()
        cp_v.wait()
        pltpu.sync_copy(qbuf, qg_hbm.at[pl.ds(base, CH)])
        pltpu.sync_copy(vbuf, vg_hbm.at[pl.ds(base, CH)])

      return carry

    lax.fori_loop(0, iters, body, 0)

  return gather_k


def _make_scatter(E, N, D):
  n_chunks = E // CH
  iters = (n_chunks + NW - 1) // NW
  mesh = plsc.VectorSubcoreMesh(core_axis_name="c", subcore_axis_name="s")

  @functools.partial(
      pl.kernel, mesh=mesh,
      out_type=jax.ShapeDtypeStruct((NC, N, D), jnp.float32),
      scratch_types=[
          pltpu.VMEM((CH,), jnp.int32),
          pltpu.VMEM((CH, D), jnp.float32),
          pltpu.VMEM_SHARED((N, D), jnp.float32),
      ])
  def scatter_k(ct_hbm, row_hbm, zero_hbm, out_hbm, idx, buf, acc):
    cid = lax.axis_index("c")
    sid = lax.axis_index("s")
    wid = sid * NC + cid

    @pl.when(sid == 0)
    def _():
      pltpu.sync_copy(zero_hbm, acc)

    plsc.subcore_barrier()

    def body(i, carry):
      c = wid + NW * i

      @pl.when(c < n_chunks)
      def _():
        base = c * CH
        pltpu.sync_copy(row_hbm.at[pl.ds(base, CH)], idx)
        pltpu.sync_copy(ct_hbm.at[pl.ds(base, CH)], buf)
        pltpu.sync_copy(buf, acc.at[idx], add=True)

      return carry

    lax.fori_loop(0, iters, body, 0)
    plsc.subcore_barrier()

    @pl.when(sid == 0)
    def _():
      pltpu.sync_copy(acc, out_hbm.at[cid])

  return scatter_k


def _mlp_body(qg, vg, w1a, w1b, w2, b1, b2, w3rep, b3, ct):
  q = qg[...].astype(jnp.bfloat16)
  v = vg[...]
  h = jnp.dot(q, w1a[...], preferred_element_type=jnp.float32)
  h += jnp.dot(v.astype(jnp.bfloat16), w1b[...],
               preferred_element_type=jnp.float32)
  h = jnp.maximum(h + b1[...], 0.0)
  h = jnp.dot(h.astype(jnp.bfloat16), w2[...],
              preferred_element_type=jnp.float32)
  h = jnp.maximum(h + b2[...], 0.0)
  logit = jnp.dot(h.astype(jnp.bfloat16), w3rep[...],
                  preferred_element_type=jnp.float32) + b3[...]
  w = 0.5 * jnp.tanh(0.5 * logit) + 0.5
  ct[...] = w * v


def _run_mlp(qg, vg, W1a, W1b, W2, b1, b2, w3rep, b3, E, D, H, TE):
  grid = (E // TE,)
  const = lambda i: (0, 0)
  return pl.pallas_call(
      _mlp_body,
      grid=grid,
      in_specs=[
          pl.BlockSpec((TE, D), lambda i: (i, 0)),
          pl.BlockSpec((TE, D), lambda i: (i, 0)),
          pl.BlockSpec((D, H), const),
          pl.BlockSpec((D, H), const),
          pl.BlockSpec((H, H), const),
          pl.BlockSpec((1, H), const),
          pl.BlockSpec((1, H), const),
          pl.BlockSpec((H, D), const),
          pl.BlockSpec((1, 1), const),
      ],
      out_specs=pl.BlockSpec((TE, D), lambda i: (i, 0)),
      out_shape=jax.ShapeDtypeStruct((E, D), jnp.float32),
  )(qg, vg, W1a, W1b, W2, b1, b2, w3rep, b3)


def _add_body(a, b, o):
  o[...] = a[...] + b[...]


def _run_add(p0, p1, N, D, TN):
  return pl.pallas_call(
      _add_body,
      grid=(N // TN,),
      in_specs=[pl.BlockSpec((TN, D), lambda i: (i, 0)),
                pl.BlockSpec((TN, D), lambda i: (i, 0))],
      out_specs=pl.BlockSpec((TN, D), lambda i: (i, 0)),
      out_shape=jax.ShapeDtypeStruct((N, D), jnp.float32),
  )(p0, p1)


def kernel(query, memory, adj_indices, W1, b1, W2, b2, W3, b3):
  N, D = query.shape
  E = adj_indices.shape[0]
  H = W2.shape[0]

  row = adj_indices[:, 0]
  col = adj_indices[:, 1]

  qg, vg = _make_gather(E, N, D)(query, memory, row, col)

  W1a = W1[:D].astype(jnp.bfloat16)
  W1b = W1[D:].astype(jnp.bfloat16)
  b1r = b1.reshape(1, H)
  b2r = b2.reshape(1, H)
  w3rep = jnp.tile(W3, (1, D)).astype(jnp.bfloat16)
  b3r = b3.reshape(1, 1)
  ct = _run_mlp(qg, vg, W1a, W1b, W2.astype(jnp.bfloat16), b1r, b2r, w3rep,
                b3r, E, D, H, TE=1280)

  zeros = jnp.zeros((N, D), jnp.float32)
  parts = _make_scatter(E, N, D)(ct, row, zeros)

  return _run_add(parts[0], parts[1], N, D, TN=2000)


# speedup vs baseline: 4.9202x; 1.5346x over previous
import functools

import jax
import jax.numpy as jnp
from jax import lax
from jax.experimental import pallas as pl
from jax.experimental.pallas import tpu as pltpu
from jax.experimental.pallas import tpu_sc as plsc

NC = 2
NS = 16
NW = NC * NS
CH = 128
NBG = 3
NBS = 2


def _make_gather(E, N, D):
  n_chunks = E // CH
  iters = (n_chunks + NW - 1) // NW
  rounds = (iters + NBG - 1) // NBG
  mesh = plsc.VectorSubcoreMesh(core_axis_name="c", subcore_axis_name="s")

  @functools.partial(
      pl.kernel, mesh=mesh,
      out_type=(jax.ShapeDtypeStruct((E, D), jnp.float32),
                jax.ShapeDtypeStruct((E, D), jnp.float32)),
      scratch_types=[
          pltpu.VMEM((NBG, CH), jnp.int32),
          pltpu.VMEM((NBG, CH), jnp.int32),
          pltpu.VMEM((NBG, CH, D), jnp.float32),
          pltpu.VMEM((NBG, CH, D), jnp.float32),
      ] + [pltpu.SemaphoreType.DMA] * (3 * NBG))
  def gather_k(query_hbm, memory_hbm, row_hbm, col_hbm, qg_hbm, vg_hbm,
               ridx, cidx, qbuf, vbuf, *sems):
    lsem = sems[0:NBG]
    gsem = sems[NBG:2 * NBG]
    osem = sems[2 * NBG:3 * NBG]
    wid = lax.axis_index("s") * NC + lax.axis_index("c")

    def fire_loads(j, b):
      c = wid + NW * j

      @pl.when(c < n_chunks)
      def _():
        base = c * CH
        pltpu.async_copy(row_hbm.at[pl.ds(base, CH)], ridx.at[b], lsem[b])
        pltpu.async_copy(col_hbm.at[pl.ds(base, CH)], cidx.at[b], lsem[b])

    for b in range(NBG):
      fire_loads(b, b)

    def body(g, carry):
      for b in range(NBG):
        j = g * NBG + b
        c = wid + NW * j

        @pl.when(c < n_chunks)
        def _(b=b, c=c):
          pltpu.make_async_copy(row_hbm.at[pl.ds(0, CH)], ridx.at[b],
                                lsem[b]).wait()
          pltpu.make_async_copy(col_hbm.at[pl.ds(0, CH)], cidx.at[b],
                                lsem[b]).wait()
          pltpu.async_copy(query_hbm.at[ridx.at[b]], qbuf.at[b], gsem[b])
          pltpu.async_copy(memory_hbm.at[cidx.at[b]], vbuf.at[b], gsem[b])

      for b in range(NBG):
        j = g * NBG + b
        c = wid + NW * j

        @pl.when(c < n_chunks)
        def _(b=b, c=c):
          base = c * CH
          pltpu.make_async_copy(query_hbm.at[ridx.at[b]], qbuf.at[b],
                                gsem[b]).wait()
          pltpu.make_async_copy(memory_hbm.at[cidx.at[b]], vbuf.at[b],
                                gsem[b]).wait()
          pltpu.async_copy(qbuf.at[b], qg_hbm.at[pl.ds(base, CH)], osem[b])
          pltpu.async_copy(vbuf.at[b], vg_hbm.at[pl.ds(base, CH)], osem[b])

      for b in range(NBG):
        j = g * NBG + b
        c = wid + NW * j

        @pl.when(c < n_chunks)
        def _(b=b, c=c):
          pltpu.make_async_copy(qbuf.at[b], qg_hbm.at[pl.ds(0, CH)],
                                osem[b]).wait()
          pltpu.make_async_copy(vbuf.at[b], vg_hbm.at[pl.ds(0, CH)],
                                osem[b]).wait()

        fire_loads((g + 1) * NBG + b, b)

      return carry

    lax.fori_loop(0, rounds, body, 0)

  return gather_k


def _make_scatter(E, N, D):
  n_chunks = E // CH
  iters = (n_chunks + NW - 1) // NW
  rounds = (iters + NBS - 1) // NBS
  mesh = plsc.VectorSubcoreMesh(core_axis_name="c", subcore_axis_name="s")

  @functools.partial(
      pl.kernel, mesh=mesh,
      out_type=jax.ShapeDtypeStruct((NC, N, D), jnp.float32),
      scratch_types=[
          pltpu.VMEM((NBS, CH), jnp.int32),
          pltpu.VMEM((NBS, CH, D), jnp.float32),
          pltpu.VMEM_SHARED((N, D), jnp.float32),
      ] + [pltpu.SemaphoreType.DMA] * (2 * NBS))
  def scatter_k(ct_hbm, row_hbm, zero_hbm, out_hbm, idx, buf, acc, *sems):
    lsem = sems[0:NBS]
    ssem = sems[NBS:2 * NBS]
    cid = lax.axis_index("c")
    sid = lax.axis_index("s")
    wid = sid * NC + cid

    @pl.when(sid == 0)
    def _():
      pltpu.sync_copy(zero_hbm, acc)

    plsc.subcore_barrier()

    def fire_loads(j, b):
      c = wid + NW * j

      @pl.when(c < n_chunks)
      def _():
        base = c * CH
        pltpu.async_copy(row_hbm.at[pl.ds(base, CH)], idx.at[b], lsem[b])
        pltpu.async_copy(ct_hbm.at[pl.ds(base, CH)], buf.at[b], lsem[b])

    for b in range(NBS):
      fire_loads(b, b)

    def body(g, carry):
      for b in range(NBS):
        j = g * NBS + b
        c = wid + NW * j

        @pl.when(c < n_chunks)
        def _(b=b, c=c):
          pltpu.make_async_copy(row_hbm.at[pl.ds(0, CH)], idx.at[b],
                                lsem[b]).wait()
          pltpu.make_async_copy(ct_hbm.at[pl.ds(0, CH)], buf.at[b],
                                lsem[b]).wait()
          pltpu.async_copy(buf.at[b], acc.at[idx.at[b]], ssem[b], add=True)

      for b in range(NBS):
        j = g * NBS + b
        c = wid + NW * j

        @pl.when(c < n_chunks)
        def _(b=b, c=c):
          pltpu.make_async_copy(buf.at[b], acc.at[idx.at[b]],
                                ssem[b]).wait()

        fire_loads((g + 1) * NBS + b, b)

      return carry

    lax.fori_loop(0, rounds, body, 0)
    plsc.subcore_barrier()

    @pl.when(sid == 0)
    def _():
      pltpu.sync_copy(acc, out_hbm.at[cid])

  return scatter_k


def _mlp_body(qg, vg, w1a, w1b, w2, b1, b2, w3rep, b3, ct):
  q = qg[...].astype(jnp.bfloat16)
  v = vg[...]
  h = jnp.dot(q, w1a[...], preferred_element_type=jnp.float32)
  h += jnp.dot(v.astype(jnp.bfloat16), w1b[...],
               preferred_element_type=jnp.float32)
  h = jnp.maximum(h + b1[...], 0.0)
  h = jnp.dot(h.astype(jnp.bfloat16), w2[...],
              preferred_element_type=jnp.float32)
  h = jnp.maximum(h + b2[...], 0.0)
  logit = jnp.dot(h.astype(jnp.bfloat16), w3rep[...],
                  preferred_element_type=jnp.float32) + b3[...]
  w = 0.5 * jnp.tanh(0.5 * logit) + 0.5
  ct[...] = w * v


def _run_mlp(qg, vg, W1a, W1b, W2, b1, b2, w3rep, b3, E, D, H, TE):
  grid = (E // TE,)
  const = lambda i: (0, 0)
  return pl.pallas_call(
      _mlp_body,
      grid=grid,
      in_specs=[
          pl.BlockSpec((TE, D), lambda i: (i, 0)),
          pl.BlockSpec((TE, D), lambda i: (i, 0)),
          pl.BlockSpec((D, H), const),
          pl.BlockSpec((D, H), const),
          pl.BlockSpec((H, H), const),
          pl.BlockSpec((1, H), const),
          pl.BlockSpec((1, H), const),
          pl.BlockSpec((H, D), const),
          pl.BlockSpec((1, 1), const),
      ],
      out_specs=pl.BlockSpec((TE, D), lambda i: (i, 0)),
      out_shape=jax.ShapeDtypeStruct((E, D), jnp.float32),
  )(qg, vg, W1a, W1b, W2, b1, b2, w3rep, b3)


def _add_body(parts, o):
  o[...] = jnp.sum(parts[...], axis=0)


def _run_add(parts, N, D, TN):
  P = parts.shape[0]
  return pl.pallas_call(
      _add_body,
      grid=(N // TN,),
      in_specs=[pl.BlockSpec((P, TN, D), lambda i: (0, i, 0))],
      out_specs=pl.BlockSpec((TN, D), lambda i: (i, 0)),
      out_shape=jax.ShapeDtypeStruct((N, D), jnp.float32),
  )(parts)


def kernel(query, memory, adj_indices, W1, b1, W2, b2, W3, b3):
  N, D = query.shape
  E = adj_indices.shape[0]
  H = W2.shape[0]

  row = adj_indices[:, 0]
  col = adj_indices[:, 1]

  W1a = W1[:D].astype(jnp.bfloat16)
  W1b = W1[D:].astype(jnp.bfloat16)
  b1r = b1.reshape(1, H)
  b2r = b2.reshape(1, H)
  w3rep = jnp.tile(W3, (1, D)).astype(jnp.bfloat16)
  b3r = b3.reshape(1, 1)
  zeros = jnp.zeros((N, D), jnp.float32)

  S = 4
  Es = E // S
  gather_k = _make_gather(Es, N, D)
  scatter_k = _make_scatter(Es, N, D)
  parts = []
  for s in range(S):
    sl = slice(s * Es, (s + 1) * Es)
    row_s = row[sl]
    qg, vg = gather_k(query, memory, row_s, col[sl])
    ct = _run_mlp(qg, vg, W1a, W1b, W2.astype(jnp.bfloat16), b1r, b2r,
                  w3rep, b3r, Es, D, H, TE=1600)
    parts.append(scatter_k(ct, row_s, zeros))

  stacked = jnp.concatenate(parts, axis=0)
  return _run_add(stacked, N, D, TN=2000)


# speedup vs baseline: 5.5035x; 1.1185x over previous
import functools

import jax
import jax.numpy as jnp
from jax import lax
from jax.experimental import pallas as pl
from jax.experimental.pallas import tpu as pltpu
from jax.experimental.pallas import tpu_sc as plsc

NC = 2
NS = 16
NW = NC * NS
CH = 128
NBG = 3
NBS = 3


def _make_gather(E, N, D):
  n_chunks = E // CH
  iters = (n_chunks + NS - 1) // NS
  rounds = (iters + NBG - 1) // NBG
  mesh = plsc.VectorSubcoreMesh(core_axis_name="c", subcore_axis_name="s")

  @functools.partial(
      pl.kernel, mesh=mesh,
      out_type=(jax.ShapeDtypeStruct((E, D), jnp.float32),
                jax.ShapeDtypeStruct((E, D), jnp.float32)),
      scratch_types=[
          pltpu.VMEM((NBG, CH), jnp.int32),
          pltpu.VMEM((NBG, CH, D), jnp.float32),
          pltpu.VMEM_SHARED((N, D), jnp.float32),
      ] + [pltpu.SemaphoreType.DMA] * (3 * NBG))
  def gather_k(query_hbm, memory_hbm, row_hbm, col_hbm, qg_hbm, vg_hbm,
               idx, buf, tab, *sems):
    lsem = sems[0:NBG]
    gsem = sems[NBG:2 * NBG]
    osem = sems[2 * NBG:3 * NBG]
    cid = lax.axis_index("c")
    sid = lax.axis_index("s")

    @pl.when((sid == 0) & (cid == 0))
    def _():
      pltpu.sync_copy(query_hbm, tab)

    @pl.when((sid == 0) & (cid == 1))
    def _():
      pltpu.sync_copy(memory_hbm, tab)

    plsc.subcore_barrier()

    def do_side(idx_hbm, out_hbm):
      def fire_loads(j, b):
        c = sid + NS * j

        @pl.when(c < n_chunks)
        def _():
          pltpu.async_copy(idx_hbm.at[pl.ds(c * CH, CH)], idx.at[b],
                           lsem[b])

      for b in range(NBG):
        fire_loads(b, b)

      def body(g, carry):
        for b in range(NBG):
          c = sid + NS * (g * NBG + b)

          @pl.when(c < n_chunks)
          def _(b=b, c=c):
            pltpu.make_async_copy(idx_hbm.at[pl.ds(0, CH)], idx.at[b],
                                  lsem[b]).wait()
            pltpu.async_copy(tab.at[idx.at[b]], buf.at[b], gsem[b])

        for b in range(NBG):
          c = sid + NS * (g * NBG + b)

          @pl.when(c < n_chunks)
          def _(b=b, c=c):
            pltpu.make_async_copy(tab.at[idx.at[b]], buf.at[b],
                                  gsem[b]).wait()
            pltpu.async_copy(buf.at[b], out_hbm.at[pl.ds(c * CH, CH)],
                             osem[b])

        for b in range(NBG):
          c = sid + NS * (g * NBG + b)

          @pl.when(c < n_chunks)
          def _(b=b, c=c):
            pltpu.make_async_copy(buf.at[b], out_hbm.at[pl.ds(0, CH)],
                                  osem[b]).wait()

          fire_loads((g + 1) * NBG + b, b)

        return carry

      lax.fori_loop(0, rounds, body, 0)

    @pl.when(cid == 0)
    def _():
      do_side(row_hbm, qg_hbm)

    @pl.when(cid == 1)
    def _():
      do_side(col_hbm, vg_hbm)

  return gather_k


def _make_scatter(E, N, D):
  n_chunks = E // CH
  iters = (n_chunks + NW - 1) // NW
  rounds = (iters + NBS - 1) // NBS
  mesh = plsc.VectorSubcoreMesh(core_axis_name="c", subcore_axis_name="s")

  @functools.partial(
      pl.kernel, mesh=mesh,
      out_type=jax.ShapeDtypeStruct((NC, N, D), jnp.float32),
      scratch_types=[
          pltpu.VMEM((NBS, CH), jnp.int32),
          pltpu.VMEM((NBS, CH, D), jnp.float32),
          pltpu.VMEM_SHARED((N, D), jnp.float32),
      ] + [pltpu.SemaphoreType.DMA] * (2 * NBS))
  def scatter_k(ct_hbm, row_hbm, zero_hbm, out_hbm, idx, buf, acc, *sems):
    lsem = sems[0:NBS]
    ssem = sems[NBS:2 * NBS]
    cid = lax.axis_index("c")
    sid = lax.axis_index("s")
    wid = sid * NC + cid

    @pl.when(sid == 0)
    def _():
      pltpu.sync_copy(zero_hbm, acc)

    plsc.subcore_barrier()

    def fire_loads(j, b):
      c = wid + NW * j

      @pl.when(c < n_chunks)
      def _():
        base = c * CH
        pltpu.async_copy(row_hbm.at[pl.ds(base, CH)], idx.at[b], lsem[b])
        pltpu.async_copy(ct_hbm.at[pl.ds(base, CH)], buf.at[b], lsem[b])

    for b in range(NBS):
      fire_loads(b, b)

    def body(g, carry):
      for b in range(NBS):
        j = g * NBS + b
        c = wid + NW * j

        @pl.when(c < n_chunks)
        def _(b=b, c=c):
          pltpu.make_async_copy(row_hbm.at[pl.ds(0, CH)], idx.at[b],
                                lsem[b]).wait()
          pltpu.make_async_copy(ct_hbm.at[pl.ds(0, CH)], buf.at[b],
                                lsem[b]).wait()
          pltpu.async_copy(buf.at[b], acc.at[idx.at[b]], ssem[b], add=True)

      for b in range(NBS):
        j = g * NBS + b
        c = wid + NW * j

        @pl.when(c < n_chunks)
        def _(b=b, c=c):
          pltpu.make_async_copy(buf.at[b], acc.at[idx.at[b]],
                                ssem[b]).wait()

        fire_loads((g + 1) * NBS + b, b)

      return carry

    lax.fori_loop(0, rounds, body, 0)
    plsc.subcore_barrier()

    @pl.when(sid == 0)
    def _():
      pltpu.sync_copy(acc, out_hbm.at[cid])

  return scatter_k


def _mlp_body(qg, vg, w1a, w1b, w2, b1, b2, w3rep, b3, ct):
  q = qg[...].astype(jnp.bfloat16)
  v = vg[...]
  h = jnp.dot(q, w1a[...], preferred_element_type=jnp.float32)
  h += jnp.dot(v.astype(jnp.bfloat16), w1b[...],
               preferred_element_type=jnp.float32)
  h = jnp.maximum(h + b1[...], 0.0)
  h = jnp.dot(h.astype(jnp.bfloat16), w2[...],
              preferred_element_type=jnp.float32)
  h = jnp.maximum(h + b2[...], 0.0)
  logit = jnp.dot(h.astype(jnp.bfloat16), w3rep[...],
                  preferred_element_type=jnp.float32) + b3[...]
  w = 0.5 * jnp.tanh(0.5 * logit) + 0.5
  ct[...] = w * v


def _run_mlp(qg, vg, W1a, W1b, W2, b1, b2, w3rep, b3, E, D, H, TE):
  grid = (E // TE,)
  const = lambda i: (0, 0)
  return pl.pallas_call(
      _mlp_body,
      grid=grid,
      in_specs=[
          pl.BlockSpec((TE, D), lambda i: (i, 0)),
          pl.BlockSpec((TE, D), lambda i: (i, 0)),
          pl.BlockSpec((D, H), const),
          pl.BlockSpec((D, H), const),
          pl.BlockSpec((H, H), const),
          pl.BlockSpec((1, H), const),
          pl.BlockSpec((1, H), const),
          pl.BlockSpec((H, D), const),
          pl.BlockSpec((1, 1), const),
      ],
      out_specs=pl.BlockSpec((TE, D), lambda i: (i, 0)),
      out_shape=jax.ShapeDtypeStruct((E, D), jnp.float32),
  )(qg, vg, W1a, W1b, W2, b1, b2, w3rep, b3)


def _add_body(parts, o):
  o[...] = jnp.sum(parts[...], axis=0)


def _run_add(parts, N, D, TN):
  P = parts.shape[0]
  return pl.pallas_call(
      _add_body,
      grid=(N // TN,),
      in_specs=[pl.BlockSpec((P, TN, D), lambda i: (0, i, 0))],
      out_specs=pl.BlockSpec((TN, D), lambda i: (i, 0)),
      out_shape=jax.ShapeDtypeStruct((N, D), jnp.float32),
  )(parts)


def kernel(query, memory, adj_indices, W1, b1, W2, b2, W3, b3):
  N, D = query.shape
  E = adj_indices.shape[0]
  H = W2.shape[0]

  row = adj_indices[:, 0]
  col = adj_indices[:, 1]

  W1a = W1[:D].astype(jnp.bfloat16)
  W1b = W1[D:].astype(jnp.bfloat16)
  b1r = b1.reshape(1, H)
  b2r = b2.reshape(1, H)
  w3rep = jnp.tile(W3, (1, D)).astype(jnp.bfloat16)
  b3r = b3.reshape(1, 1)
  zeros = jnp.zeros((N, D), jnp.float32)

  S = 4
  Es = E // S
  gather_k = _make_gather(Es, N, D)
  scatter_k = _make_scatter(Es, N, D)
  parts = []
  for s in range(S):
    sl = slice(s * Es, (s + 1) * Es)
    row_s = row[sl]
    qg, vg = gather_k(query, memory, row_s, col[sl])
    ct = _run_mlp(qg, vg, W1a, W1b, W2.astype(jnp.bfloat16), b1r, b2r,
                  w3rep, b3r, Es, D, H, TE=1600)
    parts.append(scatter_k(ct, row_s, zeros))

  stacked = jnp.concatenate(parts, axis=0)
  return _run_add(stacked, N, D, TN=2000)


# speedup vs baseline: 5.7790x; 1.0500x over previous
import functools

import jax
import jax.numpy as jnp
from jax import lax
from jax.experimental import pallas as pl
from jax.experimental.pallas import tpu as pltpu
from jax.experimental.pallas import tpu_sc as plsc

NC = 2
NS = 16
NW = NC * NS
CH = 128
NBG = 3
NBS = 3


def _make_gather(E, N, D):
  n_chunks = E // CH
  iters = (n_chunks + NS - 1) // NS
  rounds = (iters + NBG - 1) // NBG
  mesh = plsc.VectorSubcoreMesh(core_axis_name="c", subcore_axis_name="s")

  @functools.partial(
      pl.kernel, mesh=mesh,
      out_type=(jax.ShapeDtypeStruct((E, D), jnp.float32),
                jax.ShapeDtypeStruct((E, D), jnp.float32)),
      scratch_types=[
          pltpu.VMEM((NBG, CH), jnp.int32),
          pltpu.VMEM((NBG, CH, D), jnp.float32),
          pltpu.VMEM_SHARED((N, D), jnp.float32),
      ] + [pltpu.SemaphoreType.DMA] * (3 * NBG))
  def gather_k(query_hbm, memory_hbm, row_hbm, col_hbm, qg_hbm, vg_hbm,
               idx, buf, tab, *sems):
    lsem = sems[0:NBG]
    gsem = sems[NBG:2 * NBG]
    osem = sems[2 * NBG:3 * NBG]
    cid = lax.axis_index("c")
    sid = lax.axis_index("s")

    @pl.when((sid == 0) & (cid == 0))
    def _():
      pltpu.sync_copy(query_hbm, tab)

    @pl.when((sid == 0) & (cid == 1))
    def _():
      pltpu.sync_copy(memory_hbm, tab)

    plsc.subcore_barrier()

    def do_side(idx_hbm, out_hbm):
      def fire_loads(j, b):
        c = sid + NS * j

        @pl.when(c < n_chunks)
        def _():
          pltpu.async_copy(idx_hbm.at[pl.ds(c * CH, CH)], idx.at[b],
                           lsem[b])

      for b in range(NBG):
        fire_loads(b, b)

      def body(g, carry):
        for b in range(NBG):
          c = sid + NS * (g * NBG + b)

          @pl.when(c < n_chunks)
          def _(b=b, c=c):
            pltpu.make_async_copy(idx_hbm.at[pl.ds(0, CH)], idx.at[b],
                                  lsem[b]).wait()
            pltpu.async_copy(tab.at[idx.at[b]], buf.at[b], gsem[b])

        for b in range(NBG):
          c = sid + NS * (g * NBG + b)

          @pl.when(c < n_chunks)
          def _(b=b, c=c):
            pltpu.make_async_copy(tab.at[idx.at[b]], buf.at[b],
                                  gsem[b]).wait()
            pltpu.async_copy(buf.at[b], out_hbm.at[pl.ds(c * CH, CH)],
                             osem[b])

        for b in range(NBG):
          c = sid + NS * (g * NBG + b)

          @pl.when(c < n_chunks)
          def _(b=b, c=c):
            pltpu.make_async_copy(buf.at[b], out_hbm.at[pl.ds(0, CH)],
                                  osem[b]).wait()

          fire_loads((g + 1) * NBG + b, b)

        return carry

      lax.fori_loop(0, rounds, body, 0)

    @pl.when(cid == 0)
    def _():
      do_side(row_hbm, qg_hbm)

    @pl.when(cid == 1)
    def _():
      do_side(col_hbm, vg_hbm)

  return gather_k


def _make_scatter(E, N, D):
  n_chunks = E // CH
  iters = (n_chunks + NW - 1) // NW
  rounds = (iters + NBS - 1) // NBS
  mesh = plsc.VectorSubcoreMesh(core_axis_name="c", subcore_axis_name="s")

  @functools.partial(
      pl.kernel, mesh=mesh,
      out_type=jax.ShapeDtypeStruct((NC, N, D), jnp.float32),
      scratch_types=[
          pltpu.VMEM((NBS, CH), jnp.int32),
          pltpu.VMEM((NBS, CH, D), jnp.float32),
          pltpu.VMEM_SHARED((N, D), jnp.float32),
      ] + [pltpu.SemaphoreType.DMA] * (2 * NBS))
  def scatter_k(ct_hbm, row_hbm, zero_hbm, out_hbm, idx, buf, acc, *sems):
    lsem = sems[0:NBS]
    ssem = sems[NBS:2 * NBS]
    cid = lax.axis_index("c")
    sid = lax.axis_index("s")
    wid = sid * NC + cid

    @pl.when(sid == 0)
    def _():
      pltpu.sync_copy(zero_hbm, acc)

    plsc.subcore_barrier()

    def fire_loads(j, b):
      c = wid + NW * j

      @pl.when(c < n_chunks)
      def _():
        base = c * CH
        pltpu.async_copy(row_hbm.at[pl.ds(base, CH)], idx.at[b], lsem[b])
        pltpu.async_copy(ct_hbm.at[pl.ds(base, CH)], buf.at[b], lsem[b])

    for b in range(NBS):
      fire_loads(b, b)

    def body(g, carry):
      for b in range(NBS):
        j = g * NBS + b
        c = wid + NW * j

        @pl.when(c < n_chunks)
        def _(b=b, c=c):
          pltpu.make_async_copy(row_hbm.at[pl.ds(0, CH)], idx.at[b],
                                lsem[b]).wait()
          pltpu.make_async_copy(ct_hbm.at[pl.ds(0, CH)], buf.at[b],
                                lsem[b]).wait()
          pltpu.async_copy(buf.at[b], acc.at[idx.at[b]], ssem[b], add=True)

      for b in range(NBS):
        j = g * NBS + b
        c = wid + NW * j

        @pl.when(c < n_chunks)
        def _(b=b, c=c):
          pltpu.make_async_copy(buf.at[b], acc.at[idx.at[b]],
                                ssem[b]).wait()

        fire_loads((g + 1) * NBS + b, b)

      return carry

    lax.fori_loop(0, rounds, body, 0)
    plsc.subcore_barrier()

    @pl.when(sid == 0)
    def _():
      pltpu.sync_copy(acc, out_hbm.at[cid])

  return scatter_k


def _mlp_body(qg, vg, w1a, w1b, w2, b1, b2, w3rep, b3, ct):
  q = qg[...].astype(jnp.bfloat16)
  v = vg[...]
  h = jnp.dot(q, w1a[...], preferred_element_type=jnp.float32)
  h += jnp.dot(v.astype(jnp.bfloat16), w1b[...],
               preferred_element_type=jnp.float32)
  h = jnp.maximum(h + b1[...], 0.0)
  h = jnp.dot(h.astype(jnp.bfloat16), w2[...],
              preferred_element_type=jnp.float32)
  h = jnp.maximum(h + b2[...], 0.0)
  logit = jnp.dot(h.astype(jnp.bfloat16), w3rep[...],
                  preferred_element_type=jnp.float32) + b3[...]
  w = 0.5 * jnp.tanh(0.5 * logit) + 0.5
  ct[...] = w * v


def _run_mlp(qg, vg, W1a, W1b, W2, b1, b2, w3rep, b3, E, D, H, TE):
  grid = (E // TE,)
  const = lambda i: (0, 0)
  return pl.pallas_call(
      _mlp_body,
      grid=grid,
      in_specs=[
          pl.BlockSpec((TE, D), lambda i: (i, 0)),
          pl.BlockSpec((TE, D), lambda i: (i, 0)),
          pl.BlockSpec((D, H), const),
          pl.BlockSpec((D, H), const),
          pl.BlockSpec((H, H), const),
          pl.BlockSpec((1, H), const),
          pl.BlockSpec((1, H), const),
          pl.BlockSpec((H, D), const),
          pl.BlockSpec((1, 1), const),
      ],
      out_specs=pl.BlockSpec((TE, D), lambda i: (i, 0)),
      out_shape=jax.ShapeDtypeStruct((E, D), jnp.float32),
  )(qg, vg, W1a, W1b, W2, b1, b2, w3rep, b3)


def _add_body(*refs):
  o = refs[-1]
  acc = jnp.sum(refs[0][...], axis=0)
  for r in refs[1:-1]:
    acc += jnp.sum(r[...], axis=0)
  o[...] = acc


def _run_add(parts, N, D, TN):
  return pl.pallas_call(
      _add_body,
      grid=(N // TN,),
      in_specs=[pl.BlockSpec((NC, TN, D), lambda i: (0, i, 0))
                for _ in parts],
      out_specs=pl.BlockSpec((TN, D), lambda i: (i, 0)),
      out_shape=jax.ShapeDtypeStruct((N, D), jnp.float32),
  )(*parts)


def kernel(query, memory, adj_indices, W1, b1, W2, b2, W3, b3):
  N, D = query.shape
  E = adj_indices.shape[0]
  H = W2.shape[0]

  row = adj_indices[:, 0]
  col = adj_indices[:, 1]

  W1a = W1[:D].astype(jnp.bfloat16)
  W1b = W1[D:].astype(jnp.bfloat16)
  b1r = b1.reshape(1, H)
  b2r = b2.reshape(1, H)
  w3rep = jnp.tile(W3, (1, D)).astype(jnp.bfloat16)
  b3r = b3.reshape(1, 1)
  zeros = jnp.zeros((N, D), jnp.float32)

  S = 4
  Es = E // S
  gather_k = _make_gather(Es, N, D)
  scatter_k = _make_scatter(Es, N, D)
  parts = []
  for s in range(S):
    sl = slice(s * Es, (s + 1) * Es)
    row_s = row[sl]
    qg, vg = gather_k(query, memory, row_s, col[sl])
    ct = _run_mlp(qg, vg, W1a, W1b, W2.astype(jnp.bfloat16), b1r, b2r,
                  w3rep, b3r, Es, D, H, TE=1600)
    parts.append(scatter_k(ct, row_s, zeros))

  return _run_add(parts, N, D, TN=2000)


# speedup vs baseline: 5.7942x; 1.0026x over previous
import functools

import jax
import jax.numpy as jnp
from jax import lax
from jax.experimental import pallas as pl
from jax.experimental.pallas import tpu as pltpu
from jax.experimental.pallas import tpu_sc as plsc

NC = 2
NS = 16
NW = NC * NS
CH = 128
NBG = 3
NBS = 3


def _make_gather(E, N, D):
  n_chunks = E // CH
  iters = (n_chunks + NS - 1) // NS
  rounds = (iters + NBG - 1) // NBG
  mesh = plsc.VectorSubcoreMesh(core_axis_name="c", subcore_axis_name="s")

  @functools.partial(
      pl.kernel, mesh=mesh,
      out_type=(jax.ShapeDtypeStruct((E, D), jnp.float32),
                jax.ShapeDtypeStruct((E, D), jnp.float32)),
      scratch_types=[
          pltpu.VMEM((NBG, CH), jnp.int32),
          pltpu.VMEM((NBG, CH, D), jnp.float32),
          pltpu.VMEM_SHARED((N, D), jnp.float32),
      ] + [pltpu.SemaphoreType.DMA] * (3 * NBG))
  def gather_k(query_hbm, memory_hbm, row_hbm, col_hbm, qg_hbm, vg_hbm,
               idx, buf, tab, *sems):
    lsem = sems[0:NBG]
    gsem = sems[NBG:2 * NBG]
    osem = sems[2 * NBG:3 * NBG]
    cid = lax.axis_index("c")
    sid = lax.axis_index("s")

    @pl.when((sid == 0) & (cid == 0))
    def _():
      pltpu.sync_copy(query_hbm, tab)

    @pl.when((sid == 0) & (cid == 1))
    def _():
      pltpu.sync_copy(memory_hbm, tab)

    plsc.subcore_barrier()

    def do_side(idx_hbm, out_hbm):
      def fire_loads(j, b):
        c = sid + NS * j

        @pl.when(c < n_chunks)
        def _():
          pltpu.async_copy(idx_hbm.at[pl.ds(c * CH, CH)], idx.at[b],
                           lsem[b])

      for b in range(NBG):
        fire_loads(b, b)

      def body(g, carry):
        for b in range(NBG):
          c = sid + NS * (g * NBG + b)

          @pl.when(c < n_chunks)
          def _(b=b, c=c):
            pltpu.make_async_copy(idx_hbm.at[pl.ds(0, CH)], idx.at[b],
                                  lsem[b]).wait()
            pltpu.async_copy(tab.at[idx.at[b]], buf.at[b], gsem[b])

        for b in range(NBG):
          c = sid + NS * (g * NBG + b)

          @pl.when(c < n_chunks)
          def _(b=b, c=c):
            pltpu.make_async_copy(tab.at[idx.at[b]], buf.at[b],
                                  gsem[b]).wait()
            pltpu.async_copy(buf.at[b], out_hbm.at[pl.ds(c * CH, CH)],
                             osem[b])

        for b in range(NBG):
          c = sid + NS * (g * NBG + b)

          @pl.when(c < n_chunks)
          def _(b=b, c=c):
            pltpu.make_async_copy(buf.at[b], out_hbm.at[pl.ds(0, CH)],
                                  osem[b]).wait()

          fire_loads((g + 1) * NBG + b, b)

        return carry

      lax.fori_loop(0, rounds, body, 0)

    @pl.when(cid == 0)
    def _():
      do_side(row_hbm, qg_hbm)

    @pl.when(cid == 1)
    def _():
      do_side(col_hbm, vg_hbm)

  return gather_k


def _make_scatter(E, N, D):
  n_chunks = E // CH
  iters = (n_chunks + NW - 1) // NW
  rounds = (iters + NBS - 1) // NBS
  mesh = plsc.VectorSubcoreMesh(core_axis_name="c", subcore_axis_name="s")

  @functools.partial(
      pl.kernel, mesh=mesh,
      out_type=jax.ShapeDtypeStruct((NC, N, D), jnp.float32),
      scratch_types=[
          pltpu.VMEM((NBS, CH), jnp.int32),
          pltpu.VMEM((NBS, CH, D), jnp.float32),
          pltpu.VMEM_SHARED((N, D), jnp.float32),
      ] + [pltpu.SemaphoreType.DMA] * (2 * NBS))
  def scatter_k(ct_hbm, row_hbm, zero_hbm, out_hbm, idx, buf, acc, *sems):
    lsem = sems[0:NBS]
    ssem = sems[NBS:2 * NBS]
    cid = lax.axis_index("c")
    sid = lax.axis_index("s")
    wid = sid * NC + cid

    @pl.when(sid == 0)
    def _():
      pltpu.sync_copy(zero_hbm, acc)

    plsc.subcore_barrier()

    def fire_loads(j, b):
      c = wid + NW * j

      @pl.when(c < n_chunks)
      def _():
        base = c * CH
        pltpu.async_copy(row_hbm.at[pl.ds(base, CH)], idx.at[b], lsem[b])
        pltpu.async_copy(ct_hbm.at[pl.ds(base, CH)], buf.at[b], lsem[b])

    for b in range(NBS):
      fire_loads(b, b)

    def body(g, carry):
      for b in range(NBS):
        j = g * NBS + b
        c = wid + NW * j

        @pl.when(c < n_chunks)
        def _(b=b, c=c):
          pltpu.make_async_copy(row_hbm.at[pl.ds(0, CH)], idx.at[b],
                                lsem[b]).wait()
          pltpu.make_async_copy(ct_hbm.at[pl.ds(0, CH)], buf.at[b],
                                lsem[b]).wait()
          pltpu.async_copy(buf.at[b], acc.at[idx.at[b]], ssem[b], add=True)

      for b in range(NBS):
        j = g * NBS + b
        c = wid + NW * j

        @pl.when(c < n_chunks)
        def _(b=b, c=c):
          pltpu.make_async_copy(buf.at[b], acc.at[idx.at[b]],
                                ssem[b]).wait()

        fire_loads((g + 1) * NBS + b, b)

      return carry

    lax.fori_loop(0, rounds, body, 0)
    plsc.subcore_barrier()

    @pl.when(sid == 0)
    def _():
      pltpu.sync_copy(acc, out_hbm.at[cid])

  return scatter_k


def _mlp_body(qg, vg, w1a, w1b, w2, b1, b2, w3rep, b3, ct):
  q = qg[...].astype(jnp.bfloat16)
  v = vg[...]
  h = jnp.dot(q, w1a[...], preferred_element_type=jnp.float32)
  h += jnp.dot(v.astype(jnp.bfloat16), w1b[...],
               preferred_element_type=jnp.float32)
  h = jnp.maximum(h + b1[...], 0.0)
  h = jnp.dot(h.astype(jnp.bfloat16), w2[...],
              preferred_element_type=jnp.float32)
  h = jnp.maximum(h + b2[...], 0.0)
  logit = jnp.dot(h.astype(jnp.bfloat16), w3rep[...],
                  preferred_element_type=jnp.float32) + b3[...]
  w = 0.5 * jnp.tanh(0.5 * logit) + 0.5
  ct[...] = w * v


def _run_mlp(qg, vg, W1a, W1b, W2, b1, b2, w3rep, b3, E, D, H, TE):
  grid = (E // TE,)
  const = lambda i: (0, 0)
  return pl.pallas_call(
      _mlp_body,
      grid=grid,
      in_specs=[
          pl.BlockSpec((TE, D), lambda i: (i, 0)),
          pl.BlockSpec((TE, D), lambda i: (i, 0)),
          pl.BlockSpec((D, H), const),
          pl.BlockSpec((D, H), const),
          pl.BlockSpec((H, H), const),
          pl.BlockSpec((1, H), const),
          pl.BlockSpec((1, H), const),
          pl.BlockSpec((H, D), const),
          pl.BlockSpec((1, 1), const),
      ],
      out_specs=pl.BlockSpec((TE, D), lambda i: (i, 0)),
      out_shape=jax.ShapeDtypeStruct((E, D), jnp.float32),
  )(qg, vg, W1a, W1b, W2, b1, b2, w3rep, b3)


def _add_body(*refs):
  o = refs[-1]
  acc = jnp.sum(refs[0][...], axis=0)
  for r in refs[1:-1]:
    acc += jnp.sum(r[...], axis=0)
  o[...] = acc


def _run_add(parts, N, D, TN):
  return pl.pallas_call(
      _add_body,
      grid=(N // TN,),
      in_specs=[pl.BlockSpec((NC, TN, D), lambda i: (0, i, 0))
                for _ in parts],
      out_specs=pl.BlockSpec((TN, D), lambda i: (i, 0)),
      out_shape=jax.ShapeDtypeStruct((N, D), jnp.float32),
  )(*parts)


def kernel(query, memory, adj_indices, W1, b1, W2, b2, W3, b3):
  N, D = query.shape
  E = adj_indices.shape[0]
  H = W2.shape[0]

  row = adj_indices[:, 0]
  col = adj_indices[:, 1]

  W1a = W1[:D].astype(jnp.bfloat16)
  W1b = W1[D:].astype(jnp.bfloat16)
  b1r = b1.reshape(1, H)
  b2r = b2.reshape(1, H)
  w3rep = jnp.tile(W3, (1, D)).astype(jnp.bfloat16)
  b3r = b3.reshape(1, 1)
  zeros = jnp.zeros((N, D), jnp.float32)

  S = 5
  Es = E // S
  gather_k = _make_gather(Es, N, D)
  scatter_k = _make_scatter(Es, N, D)
  parts = []
  for s in range(S):
    sl = slice(s * Es, (s + 1) * Es)
    row_s = row[sl]
    qg, vg = gather_k(query, memory, row_s, col[sl])
    ct = _run_mlp(qg, vg, W1a, W1b, W2.astype(jnp.bfloat16), b1r, b2r,
                  w3rep, b3r, Es, D, H, TE=1600)
    parts.append(scatter_k(ct, row_s, zeros))

  return _run_add(parts, N, D, TN=2000)
